# baseline (device time: 79172 ns/iter reference)
import functools

import jax
import jax.numpy as jnp
from jax import lax
from jax.experimental import pallas as pl
from jax.experimental.pallas import tpu as pltpu

N_DEV = 4
SQ = 2048
D = 1024
HQ = 8
DH = 128
SCALE = 0.08838834764831843
BF16 = jnp.bfloat16
MESH = pl.DeviceIdType.MESH

_EDGES = (0, 64, 128, 256, 512, 768, 1024, 1280, 1536, 1792, 2048)
BLOCKS = tuple(zip(_EDGES[:-1], _EDGES[1:]))
N_BLK = len(BLOCKS)


def kernel(x, Wq, K_ext, V_ext, Wo):
    xf = x[0]
    wqb = Wq.astype(BF16)
    kb = jnp.transpose(K_ext[0].astype(BF16), (1, 0, 2))
    vb = jnp.transpose(V_ext[0].astype(BF16), (1, 0, 2))
    wob = Wo.astype(BF16)

    def body(x_ref, wq_ref, k_ref, v_ref, wo_ref, out_ref,
             ctx_ref, obuf, send_sems, recv_sems):
        my = lax.axis_index("i")

        def mk(i, link, target):
            r0, r1 = BLOCKS[i]
            return pltpu.make_async_remote_copy(
                src_ref=obuf.at[pl.ds(r0, r1 - r0)],
                dst_ref=obuf.at[pl.ds(r0, r1 - r0)],
                send_sem=send_sems.at[link, i], recv_sem=recv_sems.at[i],
                device_id=(target,), device_id_type=MESH,
            )

        def pair_barrier(sem):
            p_first = jnp.where(my % 2 == 0, 1, 0)
            p_second = jnp.where(my % 2 == 0, 3, 2)
            pl.semaphore_signal(sem, inc=1, device_id=(p_first,),
                                device_id_type=MESH)
            pl.semaphore_signal(sem, inc=1, device_id=(p_second,),
                                device_id_type=MESH)
            pl.semaphore_wait(sem, 2)

        pair_barrier(pltpu.get_barrier_semaphore())

        def diag_mask(sz):
            if sz == 64:
                return None
            r = lax.broadcasted_iota(jnp.int32, (sz, sz), 0)
            c = lax.broadcasted_iota(jnp.int32, (sz, sz), 1)
            return (c // 64) <= (r // 64)

        @pl.when(my == 0)
        def _():
            sends = []
            masks = {}
            for i, (r0, r1) in enumerate(BLOCKS):
                sz = r1 - r0
                if sz not in masks:
                    masks[sz] = diag_mask(sz)
                mask = masks[sz]
                ctx_ref[r0:r1, :] = jnp.dot(
                    x_ref[r0:r1, :].astype(BF16), wq_ref[...],
                    preferred_element_type=jnp.float32,
                ).astype(BF16)
                for h in range(HQ):
                    c0 = h * DH
                    q = ctx_ref[r0:r1, c0:c0 + DH]
                    s_diag = lax.dot_general(
                        q, k_ref[h, r0:r1, :],
                        (((1,), (1,)), ((), ())),
                        preferred_element_type=jnp.float32,
                    ) * SCALE
                    e_diag = jnp.exp(s_diag)
                    if mask is not None:
                        e_diag = jnp.where(mask, e_diag, 0.0)
                    den = jnp.sum(e_diag, axis=1, keepdims=True)
                    acc = jnp.dot(
                        e_diag.astype(BF16), v_ref[h, r0:r1, :],
                        preferred_element_type=jnp.float32,
                    )
                    if r0 > 0:
                        s_main = lax.dot_general(
                            q, k_ref[h, :r0, :],
                            (((1,), (1,)), ((), ())),
                            preferred_element_type=jnp.float32,
                        ) * SCALE
                        e_main = jnp.exp(s_main)
                        den = den + jnp.sum(e_main, axis=1, keepdims=True)
                        acc = acc + jnp.dot(
                            e_main.astype(BF16), v_ref[h, :r0, :],
                            preferred_element_type=jnp.float32,
                        )
                    ctx_ref[r0:r1, c0:c0 + DH] = (acc / den).astype(BF16)
                out_bf = jnp.dot(
                    ctx_ref[r0:r1, :], wo_ref[...],
                    preferred_element_type=jnp.float32,
                ).astype(BF16)
                obuf[r0:r1, :] = out_bf
                r1s = mk(i, 0, 1)
                r3s = mk(i, 1, 3)
                r1s.start()
                r3s.start()
                sends += [r1s, r3s]
                out_ref[0, r0:r1, :] = out_bf
            for r in sends:
                r.wait_send()

        @pl.when(my != 0)
        def _():
            for i, (r0, r1) in enumerate(BLOCKS):
                rcv = mk(i, 0, 0)
                rcv.wait_recv()
                relayer = (i % 2) * 2 + 1

                @pl.when(my == relayer)
                def _():
                    mk(i, 0, 2).start()

                out_ref[0, r0:r1, :] = obuf[r0:r1, :]

            @pl.when(my == 1)
            def _():
                for i in range(0, N_BLK, 2):
                    mk(i, 0, 2).wait_send()

            @pl.when(my == 3)
            def _():
                for i in range(1, N_BLK, 2):
                    mk(i, 0, 2).wait_send()

        @functools.partial(pl.run_scoped, xbar=pltpu.SemaphoreType.REGULAR)
        def _(xbar):
            pair_barrier(xbar)

    out = pl.pallas_call(
        body,
        out_shape=jax.ShapeDtypeStruct((1, SQ, D), BF16),
        in_specs=[pl.BlockSpec(memory_space=pltpu.VMEM)] * 5,
        out_specs=pl.BlockSpec(memory_space=pltpu.VMEM),
        scratch_shapes=[
            pltpu.VMEM((SQ, D), BF16),
            pltpu.VMEM((SQ, D), BF16),
            pltpu.SemaphoreType.DMA((2, N_BLK)),
            pltpu.SemaphoreType.DMA((N_BLK,)),
        ],
        compiler_params=pltpu.CompilerParams(collective_id=0),
    )(xf, wqb, kb, vb, wob)
    return out


# device time: 76258 ns/iter; 1.0382x vs baseline; 1.0382x over previous
import functools

import jax
import jax.numpy as jnp
from jax import lax
from jax.experimental import pallas as pl
from jax.experimental.pallas import tpu as pltpu

N_DEV = 4
SQ = 2048
D = 1024
HQ = 8
DH = 128
BLK = 256
N_BLK = SQ // BLK
SCALE = 0.08838834764831843
BF16 = jnp.bfloat16
MESH = pl.DeviceIdType.MESH


def kernel(x, Wq, K_ext, V_ext, Wo):
    xb = x[0].astype(BF16)
    wqb = Wq.astype(BF16)
    kb = jnp.transpose(K_ext[0].astype(BF16), (1, 0, 2))
    vb = jnp.transpose(V_ext[0].astype(BF16), (1, 0, 2))
    wob = Wo.astype(BF16)

    def body(x_ref, wq_ref, k_ref, v_ref, wo_ref, out_ref,
             ctx_ref, obuf, send_sems, recv_sems):
        my = lax.axis_index("i")

        def mk(blk, link, target):
            return pltpu.make_async_remote_copy(
                src_ref=obuf.at[blk], dst_ref=obuf.at[blk],
                send_sem=send_sems.at[link, blk], recv_sem=recv_sems.at[blk],
                device_id=(target,), device_id_type=MESH,
            )

        def pair_barrier(sem):
            p_first = jnp.where(my % 2 == 0, 1, 0)
            p_second = jnp.where(my % 2 == 0, 3, 2)
            pl.semaphore_signal(sem, inc=1, device_id=(p_first,),
                                device_id_type=MESH)
            pl.semaphore_signal(sem, inc=1, device_id=(p_second,),
                                device_id_type=MESH)
            pl.semaphore_wait(sem, 2)

        pair_barrier(pltpu.get_barrier_semaphore())

        riota = lax.broadcasted_iota(jnp.int32, (BLK, BLK), 0)
        ciota = lax.broadcasted_iota(jnp.int32, (BLK, BLK), 1)
        mask_diag = (ciota // 64) <= (riota // 64)

        @pl.when(my == 0)
        def _():
            sends = []
            for blk in range(N_BLK):
                r0 = blk * BLK
                ctx_ref[r0:r0 + BLK, :] = jnp.dot(
                    x_ref[r0:r0 + BLK, :], wq_ref[...],
                    preferred_element_type=jnp.float32,
                ).astype(BF16)
                for h in range(HQ):
                    c0 = h * DH
                    q = ctx_ref[r0:r0 + BLK, c0:c0 + DH]
                    s_diag = lax.dot_general(
                        q, k_ref[h, r0:r0 + BLK, :],
                        (((1,), (1,)), ((), ())),
                        preferred_element_type=jnp.float32,
                    ) * SCALE
                    e_diag = jnp.where(mask_diag, jnp.exp(s_diag), 0.0)
                    den = jnp.sum(e_diag, axis=1, keepdims=True)
                    acc = jnp.dot(
                        e_diag.astype(BF16), v_ref[h, r0:r0 + BLK, :],
                        preferred_element_type=jnp.float32,
                    )
                    if blk > 0:
                        s_main = lax.dot_general(
                            q, k_ref[h, :r0, :],
                            (((1,), (1,)), ((), ())),
                            preferred_element_type=jnp.float32,
                        ) * SCALE
                        e_main = jnp.exp(s_main)
                        den = den + jnp.sum(e_main, axis=1, keepdims=True)
                        acc = acc + jnp.dot(
                            e_main.astype(BF16), v_ref[h, :r0, :],
                            preferred_element_type=jnp.float32,
                        )
                    ctx_ref[r0:r0 + BLK, c0:c0 + DH] = (
                        (acc / den).astype(BF16)
                    )
                out_bf = jnp.dot(
                    ctx_ref[r0:r0 + BLK, :], wo_ref[...],
                    preferred_element_type=jnp.float32,
                ).astype(BF16)
                obuf[blk] = out_bf
                r1 = mk(blk, 0, 1)
                r3 = mk(blk, 1, 3)
                r1.start()
                r3.start()
                sends += [r1, r3]
                out_ref[0, r0:r0 + BLK, :] = out_bf
            for r in sends:
                r.wait_send()

        @pl.when(my != 0)
        def _():
            for blk in range(N_BLK):
                rcv = mk(blk, 0, 0)
                rcv.wait_recv()
                relayer = (blk % 2) * 2 + 1

                @pl.when(my == relayer)
                def _():
                    mk(blk, 0, 2).start()

                out_ref[0, blk * BLK:(blk + 1) * BLK, :] = obuf[blk]

            @pl.when(my == 1)
            def _():
                for blk in range(0, N_BLK, 2):
                    mk(blk, 0, 2).wait_send()

            @pl.when(my == 3)
            def _():
                for blk in range(1, N_BLK, 2):
                    mk(blk, 0, 2).wait_send()

        @functools.partial(pl.run_scoped, xbar=pltpu.SemaphoreType.REGULAR)
        def _(xbar):
            pair_barrier(xbar)

    out = pl.pallas_call(
        body,
        out_shape=jax.ShapeDtypeStruct((1, SQ, D), BF16),
        in_specs=[pl.BlockSpec(memory_space=pltpu.VMEM)] * 5,
        out_specs=pl.BlockSpec(memory_space=pltpu.VMEM),
        scratch_shapes=[
            pltpu.VMEM((SQ, D), BF16),
            pltpu.VMEM((N_BLK, BLK, D), BF16),
            pltpu.SemaphoreType.DMA((2, N_BLK)),
            pltpu.SemaphoreType.DMA((N_BLK,)),
        ],
        compiler_params=pltpu.CompilerParams(collective_id=0),
    )(xb, wqb, kb, vb, wob)
    return out
